# baseline (speedup 1.0000x reference)
import functools

import numpy as np

import jax
import jax.numpy as jnp
from jax import lax
from jax.experimental import pallas as pl
from jax.experimental.pallas import tpu as pltpu
from jax.experimental.pallas import tpu_sc as plsc

N_NODES = 10000
N_EDGES = 320000
D_NODE = 128
D_EDGE = 16
D_OUT = 128

NPAD = 10112
CH = 128
NC = 2
NS = 16
NW = NC * NS
CPT = 80
NCHUNK = NW * CPT
EPAD = NCHUNK * CH
RPT = NPAD // NS


def _sc_body(idxall_h, node_x_h, edge_x_h,
             accn_out, acce_out,
             ix0, ix1, ix2, ix3, rows0, rows1, erows0, erows1,
             acc_n, acc_e,
             isem0, isem1, isem2, isem3,
             gn0, gn1, ge0, ge1, sn0, sn1, se0, se1):
    cid = lax.axis_index("c")
    sid = lax.axis_index("s")
    w = sid * NC + cid
    nreal = (_NREAL // NW) + jnp.where(w < _NREAL % NW, 1, 0)
    rstart = (_NREAL // NW) * w + jnp.minimum(w, _NREAL % NW)
    doff = 2 * w - jnp.minimum(w, _NREAL % NW)

    def _src(j):
        return jnp.where(j < nreal, rstart + j, _NREAL + doff + (j - nreal))

    idx = (ix0, ix1, ix2, ix3)
    rows = (rows0, rows1)
    erows = (erows0, erows1)
    isem = (isem0, isem1, isem2, isem3)
    gsem = (gn0, gn1)
    gesem = (ge0, ge1)
    ssem = (sn0, sn1)
    sesem = (se0, se1)

    _ZERO16 = jnp.zeros((16,), jnp.float32)

    def _zero_rows(i, _):
        for k in range(D_NODE // 16):
            rows0[i, pl.ds(k * 16, 16)] = _ZERO16
        erows0[i, pl.ds(0, 16)] = _ZERO16
        return 0
    lax.fori_loop(0, CH, _zero_rows, 0)
    base = sid * RPT
    for j in range(4):
        pltpu.sync_copy(rows0, acc_n.at[pl.ds(base + j * CH, CH)])
        pltpu.sync_copy(erows0, acc_e.at[pl.ds(base + j * CH, CH)])
    pltpu.sync_copy(rows0.at[pl.ds(0, RPT - 4 * CH)],
                    acc_n.at[pl.ds(base + 4 * CH, RPT - 4 * CH)])
    pltpu.sync_copy(erows0.at[pl.ds(0, RPT - 4 * CH)],
                    acc_e.at[pl.ds(base + 4 * CH, RPT - 4 * CH)])
    plsc.subcore_barrier()

    def _idx_issue(j, k):
        pltpu.async_copy(idxall_h.at[_src(j)], idx[k], isem[k])

    def _idx_wait(j, k):
        pltpu.make_async_copy(idxall_h.at[_src(j)], idx[k], isem[k]).wait()

    def _gissue(k, p):
        pltpu.async_copy(node_x_h.at[idx[k].at[0]], rows[p], gsem[p])
        pltpu.async_copy(edge_x_h.at[idx[k].at[2]], erows[p], gesem[p])

    def _gwait(k, p):
        pltpu.make_async_copy(node_x_h.at[idx[k].at[0]], rows[p],
                              gsem[p]).wait()
        pltpu.make_async_copy(edge_x_h.at[idx[k].at[2]], erows[p],
                              gesem[p]).wait()

    def _sissue(k, p):
        pltpu.async_copy(rows[p], acc_n.at[idx[k].at[1]], ssem[p], add=True)
        pltpu.async_copy(erows[p], acc_e.at[idx[k].at[3]], sesem[p], add=True)

    def _swait(k, p):
        pltpu.make_async_copy(rows[p], acc_n.at[idx[k].at[1]],
                              ssem[p]).wait()
        pltpu.make_async_copy(erows[p], acc_e.at[idx[k].at[3]],
                              sesem[p]).wait()

    _idx_issue(0, 0)
    _idx_issue(1, 1)
    _idx_wait(0, 0)
    _gissue(0, 0)

    def _outer(i, carry):
        for b in range(4):
            j = 4 * i + b
            p = b % 2
            _gwait(b, p)
            _sissue(b, p)
            @pl.when(j + 1 < CPT)
            def _():
                _idx_wait(j + 1, (b + 1) % 4)

                @pl.when(j >= 1)
                def _():
                    _swait((b + 3) % 4, 1 - p)
                _gissue((b + 1) % 4, 1 - p)
            @pl.when(j + 2 < CPT)
            def _():
                _idx_issue(j + 2, (b + 2) % 4)
        return carry

    lax.fori_loop(0, CPT // 4, _outer, 0)
    _swait(2, 0)
    _swait(3, 1)
    plsc.subcore_barrier()

    for j in range(4):
        r = base + j * CH
        pltpu.sync_copy(acc_n.at[pl.ds(r, CH)], accn_out.at[cid, pl.ds(r, CH)])
        pltpu.sync_copy(acc_e.at[pl.ds(r, CH)], acce_out.at[cid, pl.ds(r, CH)])
    r = base + 4 * CH
    t = RPT - 4 * CH
    pltpu.sync_copy(acc_n.at[pl.ds(r, t)], accn_out.at[cid, pl.ds(r, t)])
    pltpu.sync_copy(acc_e.at[pl.ds(r, t)], acce_out.at[cid, pl.ds(r, t)])


_sc_aggregate = functools.partial(
    pl.kernel,
    out_type=(
        jax.ShapeDtypeStruct((NC, NPAD, D_NODE), jnp.float32),
        jax.ShapeDtypeStruct((NC, NPAD, D_EDGE), jnp.float32),
    ),
    mesh=plsc.VectorSubcoreMesh(core_axis_name="c", subcore_axis_name="s"),
    scratch_types=[
        pltpu.VMEM((4, CH), jnp.int32),
        pltpu.VMEM((4, CH), jnp.int32),
        pltpu.VMEM((4, CH), jnp.int32),
        pltpu.VMEM((4, CH), jnp.int32),
        pltpu.VMEM((CH, D_NODE), jnp.float32),
        pltpu.VMEM((CH, D_NODE), jnp.float32),
        pltpu.VMEM((CH, D_EDGE), jnp.float32),
        pltpu.VMEM((CH, D_EDGE), jnp.float32),
        pltpu.VMEM_SHARED((NPAD, D_NODE), jnp.float32),
        pltpu.VMEM_SHARED((NPAD, D_EDGE), jnp.float32),
    ] + [pltpu.SemaphoreType.DMA] * 12,
    compiler_params=pltpu.CompilerParams(
        use_tc_tiling_on_sc=False,
        skip_device_barrier=True,
        disable_bounds_checks=True,
    ),
)(_sc_body)


ROWS_BLK = 1000


def _tc_body(nx_ref, ap_ref0, ap_ref1, ep_ref0, ep_ref1,
             wct_ref, wnt_ref, wet_ref, b_ref, o_ref):
    x = nx_ref[...]
    a = ap_ref0[0] + ap_ref1[0]
    e = ep_ref0[0] + ep_ref1[0]
    out = (jnp.dot(x, wct_ref[...], preferred_element_type=jnp.float32)
           + jnp.dot(a, wnt_ref[...], preferred_element_type=jnp.float32)
           + jnp.dot(e, wet_ref[...], preferred_element_type=jnp.float32)
           + b_ref[...])
    nrm = jnp.sqrt(jnp.sum(out * out, axis=1, keepdims=True))
    out = out / jnp.maximum(nrm, 1e-12)
    o_ref[...] = jnp.where(out >= 0, out, 0.01 * out)


def _tc_update(node_x, aggr_p, aggr1_p, wct, wnt, wet, bias):
    grid = N_NODES // ROWS_BLK
    return pl.pallas_call(
        _tc_body,
        grid=(grid,),
        in_specs=[
            pl.BlockSpec((ROWS_BLK, D_NODE), lambda i: (i, 0)),
            pl.BlockSpec((1, ROWS_BLK, D_NODE), lambda i: (0, i, 0)),
            pl.BlockSpec((1, ROWS_BLK, D_NODE), lambda i: (1, i, 0)),
            pl.BlockSpec((1, ROWS_BLK, D_EDGE), lambda i: (0, i, 0)),
            pl.BlockSpec((1, ROWS_BLK, D_EDGE), lambda i: (1, i, 0)),
            pl.BlockSpec((D_NODE, D_OUT), lambda i: (0, 0)),
            pl.BlockSpec((D_NODE, D_OUT), lambda i: (0, 0)),
            pl.BlockSpec((D_EDGE, D_OUT), lambda i: (0, 0)),
            pl.BlockSpec((1, D_OUT), lambda i: (0, 0)),
        ],
        out_specs=pl.BlockSpec((ROWS_BLK, D_OUT), lambda i: (i, 0)),
        out_shape=jax.ShapeDtypeStruct((N_NODES, D_OUT), jnp.float32),
    )(node_x, aggr_p, aggr_p, aggr1_p, aggr1_p, wct, wnt, wet, bias)


_NREAL = N_EDGES // CH
_PAD = EPAD - N_EDGES


def _pad_edges(idx, fill):
    return jnp.concatenate([idx, fill])


def kernel(node_x, edge_index, edge_x, node_edge_index,
           node_edge_scatter_index, Wc, bc, Wn, bn, We, be):
    ar = jnp.arange(_PAD, dtype=jnp.int32)
    row = _pad_edges(edge_index[0], ar % N_NODES)
    col = _pad_edges(edge_index[1], N_NODES + ar % (NPAD - N_NODES))
    nei = _pad_edges(node_edge_index, ar % N_EDGES)
    scat = _pad_edges(node_edge_scatter_index, N_NODES + ar % (NPAD - N_NODES))
    idx_all = jnp.stack([row, col, nei, scat]).reshape(
        4, NCHUNK, CH).transpose(1, 0, 2)
    aggr_p, aggr1_p = _sc_aggregate(idx_all, node_x, edge_x)
    bias = (bc + bn + be).reshape(1, D_OUT)
    return _tc_update(node_x, aggr_p, aggr1_p, Wc.T, Wn.T, We.T, bias)


# speedup vs baseline: 1.0083x; 1.0083x over previous
import functools

import numpy as np

import jax
import jax.numpy as jnp
from jax import lax
from jax.experimental import pallas as pl
from jax.experimental.pallas import tpu as pltpu
from jax.experimental.pallas import tpu_sc as plsc

N_NODES = 10000
N_EDGES = 320000
D_NODE = 128
D_EDGE = 16
D_OUT = 128

NPAD = 10112
CH = 128
NC = 2
NS = 16
NW = NC * NS
CPT = 80
NCHUNK = NW * CPT
EPAD = NCHUNK * CH
RPT = NPAD // NS


def _sc_body(idxall_h, node_x_h, edge_x_h,
             accn_out, acce_out,
             ix0, ix1, ix2, ix3, rows0, rows1, erows0, erows1,
             acc_n, acc_e,
             isem0, isem1, isem2, isem3,
             gn0, gn1, ge0, ge1, sn0, sn1, se0, se1):
    cid = lax.axis_index("c")
    sid = lax.axis_index("s")
    w = sid * NC + cid
    nreal = (_NREAL // NW) + jnp.where(w < _NREAL % NW, 1, 0)
    rstart = (_NREAL // NW) * w + jnp.minimum(w, _NREAL % NW)
    doff = 2 * w - jnp.minimum(w, _NREAL % NW)

    def _src(j):
        return jnp.where(j < nreal, rstart + j, _NREAL + doff + (j - nreal))

    idx = (ix0, ix1, ix2, ix3)
    rows = (rows0, rows1)
    erows = (erows0, erows1)
    isem = (isem0, isem1, isem2, isem3)
    gsem = (gn0, gn1)
    gesem = (ge0, ge1)
    ssem = (sn0, sn1)
    sesem = (se0, se1)

    _ZERO16 = jnp.zeros((16,), jnp.float32)

    def _zero_rows(i, _):
        for k in range(D_NODE // 16):
            rows0[i, pl.ds(k * 16, 16)] = _ZERO16
        erows0[i, pl.ds(0, 16)] = _ZERO16
        return 0
    lax.fori_loop(0, CH, _zero_rows, 0)
    base = sid * RPT
    zt = RPT - 4 * CH
    for j in range(4):
        pltpu.async_copy(rows0, acc_n.at[pl.ds(base + j * CH, CH)], gn1)
        pltpu.async_copy(erows0, acc_e.at[pl.ds(base + j * CH, CH)], ge1)
    pltpu.async_copy(rows0.at[pl.ds(0, zt)],
                     acc_n.at[pl.ds(base + 4 * CH, zt)], gn1)
    pltpu.async_copy(erows0.at[pl.ds(0, zt)],
                     acc_e.at[pl.ds(base + 4 * CH, zt)], ge1)

    def _idx_issue(j, k):
        pltpu.async_copy(idxall_h.at[_src(j)], idx[k], isem[k])

    def _idx_wait(j, k):
        pltpu.make_async_copy(idxall_h.at[_src(j)], idx[k], isem[k]).wait()

    def _gissue(k, p):
        pltpu.async_copy(node_x_h.at[idx[k].at[0]], rows[p], gsem[p])
        pltpu.async_copy(edge_x_h.at[idx[k].at[2]], erows[p], gesem[p])

    def _gwait(k, p):
        pltpu.make_async_copy(node_x_h.at[idx[k].at[0]], rows[p],
                              gsem[p]).wait()
        pltpu.make_async_copy(edge_x_h.at[idx[k].at[2]], erows[p],
                              gesem[p]).wait()

    def _sissue(k, p):
        pltpu.async_copy(rows[p], acc_n.at[idx[k].at[1]], ssem[p], add=True)
        pltpu.async_copy(erows[p], acc_e.at[idx[k].at[3]], sesem[p], add=True)

    def _swait(k, p):
        pltpu.make_async_copy(rows[p], acc_n.at[idx[k].at[1]],
                              ssem[p]).wait()
        pltpu.make_async_copy(erows[p], acc_e.at[idx[k].at[3]],
                              sesem[p]).wait()

    _idx_issue(0, 0)
    _idx_issue(1, 1)
    for j in range(4):
        pltpu.make_async_copy(rows0, acc_n.at[pl.ds(base + j * CH, CH)],
                              gn1).wait()
        pltpu.make_async_copy(erows0, acc_e.at[pl.ds(base + j * CH, CH)],
                              ge1).wait()
    pltpu.make_async_copy(rows0.at[pl.ds(0, zt)],
                          acc_n.at[pl.ds(base + 4 * CH, zt)], gn1).wait()
    pltpu.make_async_copy(erows0.at[pl.ds(0, zt)],
                          acc_e.at[pl.ds(base + 4 * CH, zt)], ge1).wait()
    _idx_wait(0, 0)
    _gissue(0, 0)
    plsc.subcore_barrier()

    def _outer(i, carry):
        for b in range(4):
            j = 4 * i + b
            p = b % 2
            _gwait(b, p)
            _sissue(b, p)
            @pl.when(j + 1 < CPT)
            def _():
                _idx_wait(j + 1, (b + 1) % 4)

                @pl.when(j >= 1)
                def _():
                    _swait((b + 3) % 4, 1 - p)
                _gissue((b + 1) % 4, 1 - p)
            @pl.when(j + 2 < CPT)
            def _():
                _idx_issue(j + 2, (b + 2) % 4)
        return carry

    lax.fori_loop(0, CPT // 4, _outer, 0)
    _swait(2, 0)
    _swait(3, 1)
    plsc.subcore_barrier()

    f1 = pltpu.async_copy(acc_n.at[pl.ds(base, RPT)],
                          accn_out.at[cid, pl.ds(base, RPT)], gn0)
    f2 = pltpu.async_copy(acc_e.at[pl.ds(base, RPT)],
                          acce_out.at[cid, pl.ds(base, RPT)], ge0)
    f1.wait()
    f2.wait()


_sc_aggregate = functools.partial(
    pl.kernel,
    out_type=(
        jax.ShapeDtypeStruct((NC, NPAD, D_NODE), jnp.float32),
        jax.ShapeDtypeStruct((NC, NPAD, D_EDGE), jnp.float32),
    ),
    mesh=plsc.VectorSubcoreMesh(core_axis_name="c", subcore_axis_name="s"),
    scratch_types=[
        pltpu.VMEM((4, CH), jnp.int32),
        pltpu.VMEM((4, CH), jnp.int32),
        pltpu.VMEM((4, CH), jnp.int32),
        pltpu.VMEM((4, CH), jnp.int32),
        pltpu.VMEM((CH, D_NODE), jnp.float32),
        pltpu.VMEM((CH, D_NODE), jnp.float32),
        pltpu.VMEM((CH, D_EDGE), jnp.float32),
        pltpu.VMEM((CH, D_EDGE), jnp.float32),
        pltpu.VMEM_SHARED((NPAD, D_NODE), jnp.float32),
        pltpu.VMEM_SHARED((NPAD, D_EDGE), jnp.float32),
    ] + [pltpu.SemaphoreType.DMA] * 12,
    compiler_params=pltpu.CompilerParams(use_tc_tiling_on_sc=False),
)(_sc_body)


ROWS_BLK = 1000


def _tc_body(nx_ref, ap_ref0, ap_ref1, ep_ref0, ep_ref1,
             wct_ref, wnt_ref, wet_ref, b_ref, o_ref):
    x = nx_ref[...]
    a = ap_ref0[0] + ap_ref1[0]
    e = ep_ref0[0] + ep_ref1[0]
    out = (jnp.dot(x, wct_ref[...], preferred_element_type=jnp.float32)
           + jnp.dot(a, wnt_ref[...], preferred_element_type=jnp.float32)
           + jnp.dot(e, wet_ref[...], preferred_element_type=jnp.float32)
           + b_ref[...])
    nrm = jnp.sqrt(jnp.sum(out * out, axis=1, keepdims=True))
    out = out / jnp.maximum(nrm, 1e-12)
    o_ref[...] = jnp.where(out >= 0, out, 0.01 * out)


def _tc_update(node_x, aggr_p, aggr1_p, wct, wnt, wet, bias):
    grid = N_NODES // ROWS_BLK
    return pl.pallas_call(
        _tc_body,
        grid=(grid,),
        in_specs=[
            pl.BlockSpec((ROWS_BLK, D_NODE), lambda i: (i, 0)),
            pl.BlockSpec((1, ROWS_BLK, D_NODE), lambda i: (0, i, 0)),
            pl.BlockSpec((1, ROWS_BLK, D_NODE), lambda i: (1, i, 0)),
            pl.BlockSpec((1, ROWS_BLK, D_EDGE), lambda i: (0, i, 0)),
            pl.BlockSpec((1, ROWS_BLK, D_EDGE), lambda i: (1, i, 0)),
            pl.BlockSpec((D_NODE, D_OUT), lambda i: (0, 0)),
            pl.BlockSpec((D_NODE, D_OUT), lambda i: (0, 0)),
            pl.BlockSpec((D_EDGE, D_OUT), lambda i: (0, 0)),
            pl.BlockSpec((1, D_OUT), lambda i: (0, 0)),
        ],
        out_specs=pl.BlockSpec((ROWS_BLK, D_OUT), lambda i: (i, 0)),
        out_shape=jax.ShapeDtypeStruct((N_NODES, D_OUT), jnp.float32),
    )(node_x, aggr_p, aggr_p, aggr1_p, aggr1_p, wct, wnt, wet, bias)


_NREAL = N_EDGES // CH
_PAD = EPAD - N_EDGES


def _pad_edges(idx, fill):
    return jnp.concatenate([idx, fill])


def kernel(node_x, edge_index, edge_x, node_edge_index,
           node_edge_scatter_index, Wc, bc, Wn, bn, We, be):
    ar = jnp.arange(_PAD, dtype=jnp.int32)
    row = _pad_edges(edge_index[0], ar % N_NODES)
    col = _pad_edges(edge_index[1], N_NODES + ar % (NPAD - N_NODES))
    nei = _pad_edges(node_edge_index, ar % N_EDGES)
    scat = _pad_edges(node_edge_scatter_index, N_NODES + ar % (NPAD - N_NODES))
    idx_all = jnp.stack([row, col, nei, scat]).reshape(
        4, NCHUNK, CH).transpose(1, 0, 2)
    aggr_p, aggr1_p = _sc_aggregate(idx_all, node_x, edge_x)
    bias = (bc + bn + be).reshape(1, D_OUT)
    return _tc_update(node_x, aggr_p, aggr1_p, Wc.T, Wn.T, We.T, bias)


# speedup vs baseline: 1.0111x; 1.0027x over previous
import functools

import numpy as np

import jax
import jax.numpy as jnp
from jax import lax
from jax.experimental import pallas as pl
from jax.experimental.pallas import tpu as pltpu
from jax.experimental.pallas import tpu_sc as plsc

N_NODES = 10000
N_EDGES = 320000
D_NODE = 128
D_EDGE = 16
D_OUT = 128

NPAD = 10112
CH = 128
NC = 2
NS = 16
NW = NC * NS
CPT = 80
NCHUNK = NW * CPT
EPAD = NCHUNK * CH
RPT = NPAD // NS


def _sc_body(idxall_h, node_x_h, edge_x_h,
             accn_out, acce_out,
             ix0, ix1, ix2, ix3, rows0, rows1, erows0, erows1,
             acc_n, acc_e,
             isem0, isem1, isem2, isem3,
             gn0, gn1, ge0, ge1, sn0, sn1, se0, se1):
    cid = lax.axis_index("c")
    sid = lax.axis_index("s")
    w = sid * NC + cid
    nreal = (_NREAL // NW) + jnp.where(w < _NREAL % NW, 1, 0)
    rstart = (_NREAL // NW) * w + jnp.minimum(w, _NREAL % NW)
    doff = 2 * w - jnp.minimum(w, _NREAL % NW)

    def _src(j):
        return jnp.where(j < nreal, rstart + j, _NREAL + doff + (j - nreal))

    idx = (ix0, ix1, ix2, ix3)
    rows = (rows0, rows1)
    erows = (erows0, erows1)
    isem = (isem0, isem1, isem2, isem3)
    gsem = (gn0, gn1)
    gesem = (ge0, ge1)
    ssem = (sn0, sn1)
    sesem = (se0, se1)

    _ZERO16 = jnp.zeros((16,), jnp.float32)

    def _zero_rows(i, _):
        for k in range(D_NODE // 16):
            rows0[i, pl.ds(k * 16, 16)] = _ZERO16
        erows0[i, pl.ds(0, 16)] = _ZERO16
        return 0
    lax.fori_loop(0, CH, _zero_rows, 0)
    base = sid * RPT
    zt = RPT - 4 * CH
    for j in range(4):
        pltpu.async_copy(rows0, acc_n.at[pl.ds(base + j * CH, CH)], gn1)
        pltpu.async_copy(erows0, acc_e.at[pl.ds(base + j * CH, CH)], ge1)
    pltpu.async_copy(rows0.at[pl.ds(0, zt)],
                     acc_n.at[pl.ds(base + 4 * CH, zt)], gn1)
    pltpu.async_copy(erows0.at[pl.ds(0, zt)],
                     acc_e.at[pl.ds(base + 4 * CH, zt)], ge1)

    def _idx_issue(j, k):
        pltpu.async_copy(idxall_h.at[_src(j)], idx[k], isem[k])

    def _idx_wait(j, k):
        pltpu.make_async_copy(idxall_h.at[_src(j)], idx[k], isem[k]).wait()

    def _gissue(k, p):
        pltpu.async_copy(node_x_h.at[idx[k].at[0]], rows[p], gsem[p])
        pltpu.async_copy(edge_x_h.at[idx[k].at[2]], erows[p], gesem[p])

    def _gwait(k, p):
        pltpu.make_async_copy(node_x_h.at[idx[k].at[0]], rows[p],
                              gsem[p]).wait()
        pltpu.make_async_copy(edge_x_h.at[idx[k].at[2]], erows[p],
                              gesem[p]).wait()

    def _sissue(k, p):
        pltpu.async_copy(rows[p], acc_n.at[idx[k].at[1]], ssem[p], add=True)
        pltpu.async_copy(erows[p], acc_e.at[idx[k].at[3]], sesem[p], add=True)

    def _swait(k, p):
        pltpu.make_async_copy(rows[p], acc_n.at[idx[k].at[1]],
                              ssem[p]).wait()
        pltpu.make_async_copy(erows[p], acc_e.at[idx[k].at[3]],
                              sesem[p]).wait()

    _idx_issue(0, 0)
    _idx_issue(1, 1)
    for j in range(4):
        pltpu.make_async_copy(rows0, acc_n.at[pl.ds(base + j * CH, CH)],
                              gn1).wait()
        pltpu.make_async_copy(erows0, acc_e.at[pl.ds(base + j * CH, CH)],
                              ge1).wait()
    pltpu.make_async_copy(rows0.at[pl.ds(0, zt)],
                          acc_n.at[pl.ds(base + 4 * CH, zt)], gn1).wait()
    pltpu.make_async_copy(erows0.at[pl.ds(0, zt)],
                          acc_e.at[pl.ds(base + 4 * CH, zt)], ge1).wait()
    _idx_wait(0, 0)
    _gissue(0, 0)
    plsc.subcore_barrier()

    def _outer(i, carry):
        for b in range(4):
            j = 4 * i + b
            p = b % 2
            _gwait(b, p)
            _sissue(b, p)
            @pl.when(j + 1 < CPT)
            def _():
                _idx_wait(j + 1, (b + 1) % 4)

                @pl.when(j >= 1)
                def _():
                    _swait((b + 3) % 4, 1 - p)
                _gissue((b + 1) % 4, 1 - p)
            @pl.when(j + 2 < CPT)
            def _():
                _idx_issue(j + 2, (b + 2) % 4)
        return carry

    lax.fori_loop(0, CPT // 4, _outer, 0)
    _swait(2, 0)
    _swait(3, 1)
    plsc.subcore_barrier()

    f1 = pltpu.async_copy(acc_n.at[pl.ds(base, RPT)],
                          accn_out.at[cid, pl.ds(base, RPT)], gn0)
    f2 = pltpu.async_copy(acc_e.at[pl.ds(base, RPT)],
                          acce_out.at[cid, pl.ds(base, RPT)], ge0)
    f1.wait()
    f2.wait()


_sc_aggregate = functools.partial(
    pl.kernel,
    out_type=(
        jax.ShapeDtypeStruct((NC, NPAD, D_NODE), jnp.float32),
        jax.ShapeDtypeStruct((NC, NPAD, D_EDGE), jnp.float32),
    ),
    mesh=plsc.VectorSubcoreMesh(core_axis_name="c", subcore_axis_name="s"),
    scratch_types=[
        pltpu.VMEM((4, CH), jnp.int32),
        pltpu.VMEM((4, CH), jnp.int32),
        pltpu.VMEM((4, CH), jnp.int32),
        pltpu.VMEM((4, CH), jnp.int32),
        pltpu.VMEM((CH, D_NODE), jnp.float32),
        pltpu.VMEM((CH, D_NODE), jnp.float32),
        pltpu.VMEM((CH, D_EDGE), jnp.float32),
        pltpu.VMEM((CH, D_EDGE), jnp.float32),
        pltpu.VMEM_SHARED((NPAD, D_NODE), jnp.float32),
        pltpu.VMEM_SHARED((NPAD, D_EDGE), jnp.float32),
    ] + [pltpu.SemaphoreType.DMA] * 12,
    compiler_params=pltpu.CompilerParams(use_tc_tiling_on_sc=False),
)(_sc_body)


ROWS_BLK = 1000


def _tc_pre_body(nx_ref, wct_ref, b_ref, o_ref):
    o_ref[...] = jnp.dot(nx_ref[...], wct_ref[...],
                         preferred_element_type=jnp.float32) + b_ref[...]


def _tc_pre(node_x, wct, bias):
    grid = N_NODES // ROWS_BLK
    return pl.pallas_call(
        _tc_pre_body,
        grid=(grid,),
        in_specs=[
            pl.BlockSpec((ROWS_BLK, D_NODE), lambda i: (i, 0)),
            pl.BlockSpec((D_NODE, D_OUT), lambda i: (0, 0)),
            pl.BlockSpec((1, D_OUT), lambda i: (0, 0)),
        ],
        out_specs=pl.BlockSpec((ROWS_BLK, D_OUT), lambda i: (i, 0)),
        out_shape=jax.ShapeDtypeStruct((N_NODES, D_OUT), jnp.float32),
    )(node_x, wct, bias)


def _tc_body(h0_ref, ap_ref0, ap_ref1, ep_ref0, ep_ref1,
             wnt_ref, wet_ref, o_ref):
    a = ap_ref0[0] + ap_ref1[0]
    e = ep_ref0[0] + ep_ref1[0]
    out = (h0_ref[...]
           + jnp.dot(a, wnt_ref[...], preferred_element_type=jnp.float32)
           + jnp.dot(e, wet_ref[...], preferred_element_type=jnp.float32))
    nrm = jnp.sqrt(jnp.sum(out * out, axis=1, keepdims=True))
    out = out / jnp.maximum(nrm, 1e-12)
    o_ref[...] = jnp.where(out >= 0, out, 0.01 * out)


def _tc_update(h0, aggr_p, aggr1_p, wnt, wet):
    grid = N_NODES // ROWS_BLK
    return pl.pallas_call(
        _tc_body,
        grid=(grid,),
        in_specs=[
            pl.BlockSpec((ROWS_BLK, D_OUT), lambda i: (i, 0)),
            pl.BlockSpec((1, ROWS_BLK, D_NODE), lambda i: (0, i, 0)),
            pl.BlockSpec((1, ROWS_BLK, D_NODE), lambda i: (1, i, 0)),
            pl.BlockSpec((1, ROWS_BLK, D_EDGE), lambda i: (0, i, 0)),
            pl.BlockSpec((1, ROWS_BLK, D_EDGE), lambda i: (1, i, 0)),
            pl.BlockSpec((D_NODE, D_OUT), lambda i: (0, 0)),
            pl.BlockSpec((D_EDGE, D_OUT), lambda i: (0, 0)),
        ],
        out_specs=pl.BlockSpec((ROWS_BLK, D_OUT), lambda i: (i, 0)),
        out_shape=jax.ShapeDtypeStruct((N_NODES, D_OUT), jnp.float32),
    )(h0, aggr_p, aggr_p, aggr1_p, aggr1_p, wnt, wet)


_NREAL = N_EDGES // CH
_PAD = EPAD - N_EDGES


def _pad_edges(idx, fill):
    return jnp.concatenate([idx, fill])


def kernel(node_x, edge_index, edge_x, node_edge_index,
           node_edge_scatter_index, Wc, bc, Wn, bn, We, be):
    ar = jnp.arange(_PAD, dtype=jnp.int32)
    row = _pad_edges(edge_index[0], ar % N_NODES)
    col = _pad_edges(edge_index[1], N_NODES + ar % (NPAD - N_NODES))
    nei = _pad_edges(node_edge_index, ar % N_EDGES)
    scat = _pad_edges(node_edge_scatter_index, N_NODES + ar % (NPAD - N_NODES))
    idx_all = jnp.stack([row, col, nei, scat]).reshape(
        4, NCHUNK, CH).transpose(1, 0, 2)
    aggr_p, aggr1_p = _sc_aggregate(idx_all, node_x, edge_x)
    bias = (bc + bn + be).reshape(1, D_OUT)
    h0 = _tc_pre(node_x, Wc.T, bias)
    return _tc_update(h0, aggr_p, aggr1_p, Wn.T, We.T)
